# Initial kernel scaffold; baseline (speedup 1.0000x reference)
import jax
import jax.numpy as jnp
from jax.experimental import pallas as pl


def kernel(loc_data, conf_data, priors, targets):
    raise NotImplementedError("write your pallas kernel here")


# speedup vs baseline: 4.1518x; 4.1518x over previous
import functools

import jax
import jax.numpy as jnp
from jax import lax
from jax.experimental import pallas as pl
from jax.experimental.pallas import tpu as pltpu
from jax.experimental.pallas import tpu_sc as plsc

_NCLS = 21
_NTRUTH = 8
_B = 32
_P = 8732
_L = 16
_PV = (_P + _L - 1) // _L
_PPAD = _PV * _L
_CHUNK = 1456
_NCHUNK = 6
_LAST_ROWS = _P - (_NCHUNK - 1) * _CHUNK
_LN2 = 0.6931471805599453
_F32_MAX_BITS = 0x7F7FFFFF


def _splat_f(v):
    return jnp.full((_L,), v, jnp.float32)


def _splat_i(v):
    return jnp.full((_L,), v, jnp.int32)


def _ln(x):
    b = plsc.bitcast(x, jnp.int32)
    e = lax.shift_right_arithmetic(b, 23) - 127
    m = plsc.bitcast((b & 0x007FFFFF) | 0x3F800000, jnp.float32)
    big = m > 1.4142135381698608
    m = jnp.where(big, m * 0.5, m)
    e = jnp.where(big, e + 1, e)
    z = (m - 1.0) / (m + 1.0)
    z2 = z * z
    p = z2 * (1.0 / 9.0) + (1.0 / 7.0)
    p = p * z2 + 0.2
    p = p * z2 + (1.0 / 3.0)
    p = p * z2 + 1.0
    return 2.0 * z * p + e.astype(jnp.float32) * _LN2


def _mbl_body(loc_hbm, conf_hbm, pr_hbm, tg_hbm, out_hbm,
              pr_pt, bv_ref, bi_ref, r_ref, conf_buf, loc_buf, tg_buf,
              res_buf):
    b = lax.axis_index("s") * 2 + lax.axis_index("c")
    iota = lax.iota(jnp.int32, _L)

    pltpu.sync_copy(pr_hbm, pr_pt.at[pl.ds(0, _P * 4)])
    pltpu.sync_copy(tg_hbm.at[b], tg_buf.at[pl.ds(8, _NTRUTH * 5)])

    def pf_body(g, carry):
        base = (g * _L + iota) * 4
        cx = plsc.load_gather(pr_pt, [base])
        cy = plsc.load_gather(pr_pt, [base + 1])
        w = plsc.load_gather(pr_pt, [base + 2])
        h = plsc.load_gather(pr_pt, [base + 3])
        plsc.store_scatter(pr_pt, [base], cx - w * 0.5)
        plsc.store_scatter(pr_pt, [base + 1], cy - h * 0.5)
        plsc.store_scatter(pr_pt, [base + 2], cx + w * 0.5)
        plsc.store_scatter(pr_pt, [base + 3], cy + h * 0.5)
        return carry

    lax.fori_loop(0, _PV, pf_body, 0)

    bp_list = []
    for t in range(_NTRUTH):
        tx1 = plsc.load_gather(tg_buf, [_splat_i(8 + t * 5)])
        ty1 = plsc.load_gather(tg_buf, [_splat_i(8 + t * 5 + 1)])
        tx2 = plsc.load_gather(tg_buf, [_splat_i(8 + t * 5 + 2)])
        ty2 = plsc.load_gather(tg_buf, [_splat_i(8 + t * 5 + 3)])
        ta = (tx2 - tx1) * (ty2 - ty1)

        def m_body(g, carry, t=t, tx1=tx1, ty1=ty1, tx2=tx2, ty2=ty2, ta=ta):
            bpv, bpp = carry
            rows = g * _L + iota
            valid = rows < _P
            base = rows * 4
            x1 = plsc.load_gather(pr_pt, [base])
            y1 = plsc.load_gather(pr_pt, [base + 1])
            x2 = plsc.load_gather(pr_pt, [base + 2])
            y2 = plsc.load_gather(pr_pt, [base + 3])
            ap = (x2 - x1) * (y2 - y1)
            ix = jnp.minimum(x2, tx2) - jnp.maximum(x1, tx1)
            iy = jnp.minimum(y2, ty2) - jnp.maximum(y1, ty1)
            inter = jnp.maximum(ix, 0.0) * jnp.maximum(iy, 0.0)
            iou = inter / (ta + ap - inter)
            iou = jnp.where(valid, iou, -1.0)
            sl = pl.ds(g * _L, _L)
            if t == 0:
                bv_ref[sl] = iou
                bi_ref[sl] = jnp.zeros((_L,), jnp.int32)
            else:
                bvo = bv_ref[sl]
                bio = bi_ref[sl]
                upd = iou > bvo
                bv_ref[sl] = jnp.where(upd, iou, bvo)
                bi_ref[sl] = jnp.where(upd, t, bio)
            u2 = iou > bpv
            bpv = jnp.where(u2, iou, bpv)
            bpp = jnp.where(u2, rows, bpp)
            return bpv, bpp

        bpv, bpp = lax.fori_loop(0, _PV, m_body,
                                 (_splat_f(-2.0), jnp.zeros((_L,), jnp.int32)))
        mv = jnp.max(bpv)
        cand = jnp.where(bpv == mv, bpp, jnp.int32(1 << 30))
        bp_list.append(jnp.min(cand))

    ll_acc = _splat_f(0.0)
    spce = _splat_f(0.0)
    npos = jnp.zeros((_L,), jnp.int32)
    for ci in range(_NCHUNK):
        cb = ci * _CHUNK
        rows_dma = _CHUNK if ci < _NCHUNK - 1 else _LAST_ROWS
        pltpu.sync_copy(conf_hbm.at[b, pl.ds(cb * _NCLS, rows_dma * _NCLS)],
                        conf_buf.at[pl.ds(0, rows_dma * _NCLS)])
        pltpu.sync_copy(loc_hbm.at[b, pl.ds(cb * 4, rows_dma * 4)],
                        loc_buf.at[pl.ds(0, rows_dma * 4)])

        def b_body(g, carry, cb=cb):
            ll, sp, npn = carry
            lrows = g * _L + iota
            rows = cb + lrows
            valid = rows < _P
            sl = pl.ds(cb + g * _L, _L)
            bv = bv_ref[sl]
            bi = bi_ref[sl]
            for t in range(_NTRUTH):
                m = rows == bp_list[t]
                bv = jnp.where(m, 2.0, bv)
                bi = jnp.where(m, t, bi)
            bi5 = bi * 5 + 8
            lab = plsc.load_gather(tg_buf, [bi5 + 4])
            conf_t = jnp.where(bv < 0.5, 0, lab.astype(jnp.int32) + 1)
            pos = (conf_t > 0) & valid
            npn = npn + jnp.where(pos, 1, 0)
            base = rows * 4
            x1 = plsc.load_gather(pr_pt, [base])
            y1 = plsc.load_gather(pr_pt, [base + 1])
            x2 = plsc.load_gather(pr_pt, [base + 2])
            y2 = plsc.load_gather(pr_pt, [base + 3])
            mx1 = plsc.load_gather(tg_buf, [bi5])
            my1 = plsc.load_gather(tg_buf, [bi5 + 1])
            mx2 = plsc.load_gather(tg_buf, [bi5 + 2])
            my2 = plsc.load_gather(tg_buf, [bi5 + 3])
            enc = (
                ((mx1 + mx2) * 0.5 - x1) / (0.1 * x2),
                ((my1 + my2) * 0.5 - y1) / (0.1 * y2),
                _ln((mx2 - mx1) / x2) * 5.0,
                _ln((my2 - my1) / y2) * 5.0,
            )
            s = _splat_f(0.0)
            lbase = lrows * 4
            for a in range(4):
                ld = plsc.load_gather(loc_buf, [lbase + a])
                d = ld - enc[a]
                ad = jnp.abs(d)
                s = s + jnp.where(ad < 1.0, 0.5 * d * d, ad - 0.5)
            ll = ll + jnp.where(pos, s, 0.0)
            cbase = lrows * _NCLS
            mx = _splat_f(-3.4e38)
            for j in range(_NCLS):
                c = plsc.load_gather(conf_buf, [cbase + j])
                mx = jnp.maximum(mx, c)
            ssum = _splat_f(0.0)
            gat = _splat_f(0.0)
            for j in range(_NCLS):
                c = plsc.load_gather(conf_buf, [cbase + j])
                ssum = ssum + jnp.exp(c - mx)
                gat = jnp.where(conf_t == j, c, gat)
            ce = _ln(ssum) + mx - gat
            sp = sp + jnp.where(pos, ce, 0.0)
            r_ref[sl] = jnp.where(pos | ~valid, 0.0, ce)
            return ll, sp, npn

        ll_acc, spce, npos = lax.fori_loop(0, _CHUNK // _L, b_body,
                                           (ll_acc, spce, npos))

    num_pos = jnp.sum(npos)
    k = jnp.minimum(3 * num_pos, _P - 1)

    def bs_body(i, lohi):
        lo, hi = lohi
        mid = lo + lax.shift_right_arithmetic(hi - lo + 1, 1)

        def cnt_body(v, c, mid=mid):
            bits = plsc.bitcast(r_ref[pl.ds(v * _L, _L)], jnp.int32)
            return c + plsc.all_reduce_population_count(bits >= mid)

        cnt = lax.fori_loop(0, _PV, cnt_body, jnp.zeros((_L,), jnp.int32))
        take = cnt >= k
        return jnp.where(take, mid, lo), jnp.where(take, hi, mid - 1)

    lo, hi = lax.fori_loop(0, 31, bs_body,
                           (jnp.zeros((_L,), jnp.int32),
                            _splat_i(_F32_MAX_BITS)))
    thr = plsc.bitcast(lo, jnp.float32)

    def tally_body(v, carry):
        cg, sg = carry
        rv = r_ref[pl.ds(v * _L, _L)]
        m = rv > thr
        return (cg + plsc.all_reduce_population_count(m),
                sg + jnp.where(m, rv, 0.0))

    cnt_gt, sum_gt = lax.fori_loop(0, _PV, tally_body,
                                   (jnp.zeros((_L,), jnp.int32),
                                    _splat_f(0.0)))
    extra = (k - cnt_gt).astype(jnp.float32)
    lc_vec = spce + sum_gt + jnp.where(iota == 0, extra * thr, 0.0)

    res_buf[pl.ds(0, _L)] = ll_acc
    res_buf[pl.ds(_L, _L)] = lc_vec
    res_buf[pl.ds(2 * _L, _L)] = npos.astype(jnp.float32)
    pltpu.sync_copy(res_buf, out_hbm.at[b])


_mbl = functools.partial(
    pl.kernel,
    out_type=jax.ShapeDtypeStruct((_B, 3 * _L), jnp.float32),
    mesh=plsc.VectorSubcoreMesh(core_axis_name="c", subcore_axis_name="s"),
    compiler_params=pltpu.CompilerParams(use_tc_tiling_on_sc=False,
                                         needs_layout_passes=False),
    scratch_types=[
        pltpu.VMEM((_PPAD * 4,), jnp.float32),
        pltpu.VMEM((_PPAD,), jnp.float32),
        pltpu.VMEM((_PPAD,), jnp.int32),
        pltpu.VMEM((_PPAD,), jnp.float32),
        pltpu.VMEM((_CHUNK * _NCLS,), jnp.float32),
        pltpu.VMEM((_CHUNK * 4,), jnp.float32),
        pltpu.VMEM((_NTRUTH * 5 + 8,), jnp.float32),
        pltpu.VMEM((3 * _L,), jnp.float32),
    ],
)(_mbl_body)


def kernel(loc_data, conf_data, priors, targets):
    out = _mbl(loc_data.reshape(_B, _P * 4),
               conf_data.reshape(_B, _P * _NCLS),
               priors.reshape(_P * 4),
               targets.reshape(_B, _NTRUTH * 5))
    sums = jnp.sum(out.reshape(_B, 3, _L), axis=(0, 2))
    n = sums[2]
    return sums[0] / n, sums[1] / n


# speedup vs baseline: 5.4807x; 1.3201x over previous
import functools

import jax
import jax.numpy as jnp
from jax import lax
from jax.experimental import pallas as pl
from jax.experimental.pallas import tpu as pltpu
from jax.experimental.pallas import tpu_sc as plsc

_NCLS = 21
_NTRUTH = 8
_B = 32
_P = 8732
_L = 16
_PV = (_P + _L - 1) // _L
_PPAD = _PV * _L
_CHUNK = 272
_NCHUNK = 32
_MAIN = _NCHUNK * _CHUNK
_TAIL = _P - _MAIN
_LN2 = 0.6931471805599453
_F32_MAX_BITS = 0x7F7FFFFF


def _splat_f(v):
    return jnp.full((_L,), v, jnp.float32)


def _splat_i(v):
    return jnp.full((_L,), v, jnp.int32)


def _ln(x):
    b = plsc.bitcast(x, jnp.int32)
    e = lax.shift_right_arithmetic(b, 23) - 127
    m = plsc.bitcast((b & 0x007FFFFF) | 0x3F800000, jnp.float32)
    big = m > 1.4142135381698608
    m = jnp.where(big, m * 0.5, m)
    e = jnp.where(big, e + 1, e)
    z = (m - 1.0) / (m + 1.0)
    z2 = z * z
    p = z2 * (1.0 / 9.0) + (1.0 / 7.0)
    p = p * z2 + 0.2
    p = p * z2 + (1.0 / 3.0)
    p = p * z2 + 1.0
    return 2.0 * z * p + e.astype(jnp.float32) * _LN2


def _mbl_body(loc_hbm, conf_hbm, ctail_hbm, pr_hbm, tg_hbm, out_hbm,
              pr_buf, bv_ref, bi_ref, r_ref, conf_buf, loc_buf, ct_buf,
              lt_buf, tg_buf, res_buf):
    b = lax.axis_index("s") * 2 + lax.axis_index("c")
    iota = lax.iota(jnp.int32, _L)

    pltpu.sync_copy(pr_hbm, pr_buf)
    pltpu.sync_copy(tg_hbm.at[b], tg_buf.at[pl.ds(8, _NTRUTH)])
    pltpu.sync_copy(ctail_hbm.at[b], ct_buf)
    pltpu.sync_copy(loc_hbm.at[pl.ds(b * (_P * 4) + _MAIN * 4, _TAIL * 4)],
                    lt_buf)

    def pf_body(g, carry):
        rows = jnp.minimum(g * _L + iota, _P - 1)
        base = rows * 4
        cx = plsc.load_gather(pr_buf, [base])
        cy = plsc.load_gather(pr_buf, [base + 1])
        w = plsc.load_gather(pr_buf, [base + 2])
        h = plsc.load_gather(pr_buf, [base + 3])
        plsc.store_scatter(pr_buf, [base], cx - w * 0.5)
        plsc.store_scatter(pr_buf, [base + 1], cy - h * 0.5)
        plsc.store_scatter(pr_buf, [base + 2], cx + w * 0.5)
        plsc.store_scatter(pr_buf, [base + 3], cy + h * 0.5)
        return carry

    lax.fori_loop(0, _PV, pf_body, 0)

    bp_list = []
    for t in range(_NTRUTH):
        tx1 = plsc.load_gather(tg_buf, [_splat_i(8 + t), _splat_i(0)])
        ty1 = plsc.load_gather(tg_buf, [_splat_i(8 + t), _splat_i(1)])
        tx2 = plsc.load_gather(tg_buf, [_splat_i(8 + t), _splat_i(2)])
        ty2 = plsc.load_gather(tg_buf, [_splat_i(8 + t), _splat_i(3)])
        ta = (tx2 - tx1) * (ty2 - ty1)

        def m_body(g, carry, t=t, tx1=tx1, ty1=ty1, tx2=tx2, ty2=ty2, ta=ta):
            bpv, bpp = carry
            rows = g * _L + iota
            valid = rows < _P
            base = jnp.minimum(rows, _P - 1) * 4
            x1 = plsc.load_gather(pr_buf, [base])
            y1 = plsc.load_gather(pr_buf, [base + 1])
            x2 = plsc.load_gather(pr_buf, [base + 2])
            y2 = plsc.load_gather(pr_buf, [base + 3])
            ap = (x2 - x1) * (y2 - y1)
            ix = jnp.minimum(x2, tx2) - jnp.maximum(x1, tx1)
            iy = jnp.minimum(y2, ty2) - jnp.maximum(y1, ty1)
            inter = jnp.maximum(ix, 0.0) * jnp.maximum(iy, 0.0)
            iou = inter / (ta + ap - inter)
            iou = jnp.where(valid, iou, -1.0)
            sl = pl.ds(g * _L, _L)
            if t == 0:
                bv_ref[sl] = iou
                bi_ref[sl] = jnp.zeros((_L,), jnp.int32)
            else:
                bvo = bv_ref[sl]
                bio = bi_ref[sl]
                upd = iou > bvo
                bv_ref[sl] = jnp.where(upd, iou, bvo)
                bi_ref[sl] = jnp.where(upd, t, bio)
            u2 = iou > bpv
            bpv = jnp.where(u2, iou, bpv)
            bpp = jnp.where(u2, rows, bpp)
            return bpv, bpp

        bpv, bpp = lax.fori_loop(0, _PV, m_body,
                                 (_splat_f(-2.0), jnp.zeros((_L,), jnp.int32)))
        mv = jnp.max(bpv)
        cand = jnp.where(bpv == mv, bpp, jnp.int32(1 << 30))
        bp_list.append(jnp.min(cand))

    ll_acc = _splat_f(0.0)
    spce = _splat_f(0.0)
    npos = jnp.zeros((_L,), jnp.int32)

    def make_chunk(cb, cf_ref, lc_ref, nvec, tail):
        def b_body(g, carry):
            ll, sp, npn = carry
            lrows = g * _L + iota
            rows = cb + lrows
            valid = rows < _P
            sl = pl.ds(cb + g * _L, _L)
            bv = bv_ref[sl]
            bi = bi_ref[sl]
            for t in range(_NTRUTH):
                m = rows == bp_list[t]
                bv = jnp.where(m, 2.0, bv)
                bi = jnp.where(m, t, bi)
            lab = plsc.load_gather(tg_buf, [bi + 8, _splat_i(4)])
            conf_t = jnp.where(bv < 0.5, 0, lab.astype(jnp.int32) + 1)
            pos = (conf_t > 0) & valid
            npn = npn + jnp.where(pos, 1, 0)
            base = jnp.minimum(rows, _P - 1) * 4
            x1 = plsc.load_gather(pr_buf, [base])
            y1 = plsc.load_gather(pr_buf, [base + 1])
            x2 = plsc.load_gather(pr_buf, [base + 2])
            y2 = plsc.load_gather(pr_buf, [base + 3])
            mx1 = plsc.load_gather(tg_buf, [bi + 8, _splat_i(0)])
            my1 = plsc.load_gather(tg_buf, [bi + 8, _splat_i(1)])
            mx2 = plsc.load_gather(tg_buf, [bi + 8, _splat_i(2)])
            my2 = plsc.load_gather(tg_buf, [bi + 8, _splat_i(3)])
            enc = (
                ((mx1 + mx2) * 0.5 - x1) / (0.1 * x2),
                ((my1 + my2) * 0.5 - y1) / (0.1 * y2),
                _ln((mx2 - mx1) / x2) * 5.0,
                _ln((my2 - my1) / y2) * 5.0,
            )
            if tail:
                crow = jnp.minimum(lrows, _TAIL - 1)
            else:
                crow = lrows
            s = _splat_f(0.0)
            lbase = crow * 4
            for a in range(4):
                ld = plsc.load_gather(lc_ref, [lbase + a])
                d = ld - enc[a]
                ad = jnp.abs(d)
                s = s + jnp.where(ad < 1.0, 0.5 * d * d, ad - 0.5)
            ll = ll + jnp.where(pos, s, 0.0)
            mx = _splat_f(-3.4e38)
            for j in range(_NCLS):
                c = plsc.load_gather(cf_ref, [crow, _splat_i(j)])
                mx = jnp.maximum(mx, c)
            ssum = _splat_f(0.0)
            gat = _splat_f(0.0)
            for j in range(_NCLS):
                c = plsc.load_gather(cf_ref, [crow, _splat_i(j)])
                ssum = ssum + jnp.exp(c - mx)
                gat = jnp.where(conf_t == j, c, gat)
            ce = _ln(ssum) + mx - gat
            sp = sp + jnp.where(pos, ce, 0.0)
            r_ref[sl] = jnp.where(pos | ~valid, 0.0, ce)
            return ll, sp, npn

        return b_body

    def chunk_body(ci, carry):
        cb = pl.multiple_of(ci * _CHUNK, _CHUNK)
        pltpu.sync_copy(conf_hbm.at[b, pl.ds(cb, _CHUNK)], conf_buf)
        pltpu.sync_copy(loc_hbm.at[pl.ds(b * (_P * 4) + cb * 4, _CHUNK * 4)],
                        loc_buf)
        return lax.fori_loop(0, _CHUNK // _L,
                             make_chunk(cb, conf_buf, loc_buf, _CHUNK // _L,
                                        False),
                             carry)

    ll_acc, spce, npos = lax.fori_loop(0, _NCHUNK, chunk_body,
                                       (ll_acc, spce, npos))

    tail_fn = make_chunk(_MAIN, ct_buf, lt_buf, 2, True)
    ll_acc, spce, npos = tail_fn(0, (ll_acc, spce, npos))
    ll_acc, spce, npos = tail_fn(1, (ll_acc, spce, npos))

    num_pos = jnp.sum(npos)
    k = jnp.minimum(3 * num_pos, _P - 1)

    def bs_body(i, lohi):
        lo, hi = lohi
        mid = lo + lax.shift_right_arithmetic(hi - lo + 1, 1)

        def cnt_body(v, c, mid=mid):
            bits = plsc.bitcast(r_ref[pl.ds(v * _L, _L)], jnp.int32)
            return c + plsc.all_reduce_population_count(bits >= mid)

        cnt = lax.fori_loop(0, _PV, cnt_body, jnp.zeros((_L,), jnp.int32))
        take = cnt >= k
        return jnp.where(take, mid, lo), jnp.where(take, hi, mid - 1)

    lo, hi = lax.fori_loop(0, 31, bs_body,
                           (jnp.zeros((_L,), jnp.int32),
                            _splat_i(_F32_MAX_BITS)))
    thr = plsc.bitcast(lo, jnp.float32)

    def tally_body(v, carry):
        cg, sg = carry
        rv = r_ref[pl.ds(v * _L, _L)]
        m = rv > thr
        return (cg + plsc.all_reduce_population_count(m),
                sg + jnp.where(m, rv, 0.0))

    cnt_gt, sum_gt = lax.fori_loop(0, _PV, tally_body,
                                   (jnp.zeros((_L,), jnp.int32),
                                    _splat_f(0.0)))
    extra = (k - cnt_gt).astype(jnp.float32)
    lc_vec = spce + sum_gt + jnp.where(iota == 0, extra * thr, 0.0)

    res_buf[0, pl.ds(0, _L)] = ll_acc
    res_buf[0, pl.ds(_L, _L)] = lc_vec
    res_buf[0, pl.ds(2 * _L, _L)] = npos.astype(jnp.float32)
    pltpu.sync_copy(res_buf, out_hbm.at[b])


_mbl = functools.partial(
    pl.kernel,
    out_type=jax.ShapeDtypeStruct((_B, 1, 3 * _L), jnp.float32),
    mesh=plsc.VectorSubcoreMesh(core_axis_name="c", subcore_axis_name="s"),
    compiler_params=pltpu.CompilerParams(use_tc_tiling_on_sc=True,
                                         needs_layout_passes=False),
    scratch_types=[
        pltpu.VMEM((_P * 4,), jnp.float32),
        pltpu.VMEM((_PPAD,), jnp.float32),
        pltpu.VMEM((_PPAD,), jnp.int32),
        pltpu.VMEM((_PPAD,), jnp.float32),
        pltpu.VMEM((_CHUNK, _NCLS), jnp.float32),
        pltpu.VMEM((_CHUNK * 4,), jnp.float32),
        pltpu.VMEM((_TAIL, _NCLS), jnp.float32),
        pltpu.VMEM((_TAIL * 4,), jnp.float32),
        pltpu.VMEM((8 + _NTRUTH, 5), jnp.float32),
        pltpu.VMEM((1, 3 * _L), jnp.float32),
    ],
)(_mbl_body)


def kernel(loc_data, conf_data, priors, targets):
    out = _mbl(loc_data.reshape(_B * _P * 4),
               conf_data,
               conf_data[:, _MAIN:, :],
               priors.reshape(_P * 4),
               targets)
    sums = jnp.sum(out.reshape(_B, 3, _L), axis=(0, 2))
    n = sums[2]
    return sums[0] / n, sums[1] / n


# speedup vs baseline: 5.4868x; 1.0011x over previous
import functools

import jax
import jax.numpy as jnp
from jax import lax
from jax.experimental import pallas as pl
from jax.experimental.pallas import tpu as pltpu
from jax.experimental.pallas import tpu_sc as plsc

_NCLS = 21
_NTRUTH = 8
_B = 32
_P = 8732
_L = 16
_PV = (_P + _L - 1) // _L
_PPAD = _PV * _L
_CHUNK = 272
_NCHUNK = 32
_MAIN = _NCHUNK * _CHUNK
_TAIL = _P - _MAIN
_LN2 = 0.6931471805599453
_F32_MAX_BITS = 0x7F7FFFFF


def _splat_f(v):
    return jnp.full((_L,), v, jnp.float32)


def _splat_i(v):
    return jnp.full((_L,), v, jnp.int32)


def _ln(x):
    b = plsc.bitcast(x, jnp.int32)
    e = lax.shift_right_arithmetic(b, 23) - 127
    m = plsc.bitcast((b & 0x007FFFFF) | 0x3F800000, jnp.float32)
    big = m > 1.4142135381698608
    m = jnp.where(big, m * 0.5, m)
    e = jnp.where(big, e + 1, e)
    z = (m - 1.0) / (m + 1.0)
    z2 = z * z
    p = z2 * (1.0 / 9.0) + (1.0 / 7.0)
    p = p * z2 + 0.2
    p = p * z2 + (1.0 / 3.0)
    p = p * z2 + 1.0
    return 2.0 * z * p + e.astype(jnp.float32) * _LN2


def _mbl_body(loc_hbm, conf_hbm, pr_hbm, tg_hbm, out_hbm,
              pr_buf, bv_ref, bi_ref, r_ref, conf_buf, loc_buf, ct_buf,
              lt_buf, tg_buf, res_buf):
    b = lax.axis_index("s") * 2 + lax.axis_index("c")
    iota = lax.iota(jnp.int32, _L)

    pltpu.sync_copy(pr_hbm, pr_buf)
    pltpu.sync_copy(tg_hbm.at[b], tg_buf.at[pl.ds(8, _NTRUTH)])
    pltpu.sync_copy(conf_hbm.at[b, pl.ds(_MAIN, _TAIL)], ct_buf)
    pltpu.sync_copy(loc_hbm.at[pl.ds(b * (_P * 4) + _MAIN * 4, _TAIL * 4)],
                    lt_buf)

    def pf_body(g, carry):
        rows = jnp.minimum(g * _L + iota, _P - 1)
        base = rows * 4
        cx = plsc.load_gather(pr_buf, [base])
        cy = plsc.load_gather(pr_buf, [base + 1])
        w = plsc.load_gather(pr_buf, [base + 2])
        h = plsc.load_gather(pr_buf, [base + 3])
        plsc.store_scatter(pr_buf, [base], cx - w * 0.5)
        plsc.store_scatter(pr_buf, [base + 1], cy - h * 0.5)
        plsc.store_scatter(pr_buf, [base + 2], cx + w * 0.5)
        plsc.store_scatter(pr_buf, [base + 3], cy + h * 0.5)
        return carry

    lax.fori_loop(0, _PV, pf_body, 0)

    bp_list = []
    for t in range(_NTRUTH):
        tx1 = plsc.load_gather(tg_buf, [_splat_i(8 + t), _splat_i(0)])
        ty1 = plsc.load_gather(tg_buf, [_splat_i(8 + t), _splat_i(1)])
        tx2 = plsc.load_gather(tg_buf, [_splat_i(8 + t), _splat_i(2)])
        ty2 = plsc.load_gather(tg_buf, [_splat_i(8 + t), _splat_i(3)])
        ta = (tx2 - tx1) * (ty2 - ty1)

        def m_body(g, carry, t=t, tx1=tx1, ty1=ty1, tx2=tx2, ty2=ty2, ta=ta):
            bpv, bpp = carry
            rows = g * _L + iota
            valid = rows < _P
            base = jnp.minimum(rows, _P - 1) * 4
            x1 = plsc.load_gather(pr_buf, [base])
            y1 = plsc.load_gather(pr_buf, [base + 1])
            x2 = plsc.load_gather(pr_buf, [base + 2])
            y2 = plsc.load_gather(pr_buf, [base + 3])
            ap = (x2 - x1) * (y2 - y1)
            ix = jnp.minimum(x2, tx2) - jnp.maximum(x1, tx1)
            iy = jnp.minimum(y2, ty2) - jnp.maximum(y1, ty1)
            inter = jnp.maximum(ix, 0.0) * jnp.maximum(iy, 0.0)
            iou = inter / (ta + ap - inter)
            iou = jnp.where(valid, iou, -1.0)
            sl = pl.ds(g * _L, _L)
            if t == 0:
                bv_ref[sl] = iou
                bi_ref[sl] = jnp.zeros((_L,), jnp.int32)
            else:
                bvo = bv_ref[sl]
                bio = bi_ref[sl]
                upd = iou > bvo
                bv_ref[sl] = jnp.where(upd, iou, bvo)
                bi_ref[sl] = jnp.where(upd, t, bio)
            u2 = iou > bpv
            bpv = jnp.where(u2, iou, bpv)
            bpp = jnp.where(u2, rows, bpp)
            return bpv, bpp

        bpv, bpp = lax.fori_loop(0, _PV, m_body,
                                 (_splat_f(-2.0), jnp.zeros((_L,), jnp.int32)))
        mv = jnp.max(bpv)
        cand = jnp.where(bpv == mv, bpp, jnp.int32(1 << 30))
        bp_list.append(jnp.min(cand))

    ll_acc = _splat_f(0.0)
    spce = _splat_f(0.0)
    npos = jnp.zeros((_L,), jnp.int32)

    def make_chunk(cb, cf_ref, lc_ref, nvec, tail):
        def b_body(g, carry):
            ll, sp, npn = carry
            lrows = g * _L + iota
            rows = cb + lrows
            valid = rows < _P
            sl = pl.ds(cb + g * _L, _L)
            bv = bv_ref[sl]
            bi = bi_ref[sl]
            for t in range(_NTRUTH):
                m = rows == bp_list[t]
                bv = jnp.where(m, 2.0, bv)
                bi = jnp.where(m, t, bi)
            lab = plsc.load_gather(tg_buf, [bi + 8, _splat_i(4)])
            conf_t = jnp.where(bv < 0.5, 0, lab.astype(jnp.int32) + 1)
            pos = (conf_t > 0) & valid
            npn = npn + jnp.where(pos, 1, 0)
            base = jnp.minimum(rows, _P - 1) * 4
            x1 = plsc.load_gather(pr_buf, [base])
            y1 = plsc.load_gather(pr_buf, [base + 1])
            x2 = plsc.load_gather(pr_buf, [base + 2])
            y2 = plsc.load_gather(pr_buf, [base + 3])
            mx1 = plsc.load_gather(tg_buf, [bi + 8, _splat_i(0)])
            my1 = plsc.load_gather(tg_buf, [bi + 8, _splat_i(1)])
            mx2 = plsc.load_gather(tg_buf, [bi + 8, _splat_i(2)])
            my2 = plsc.load_gather(tg_buf, [bi + 8, _splat_i(3)])
            enc = (
                ((mx1 + mx2) * 0.5 - x1) / (0.1 * x2),
                ((my1 + my2) * 0.5 - y1) / (0.1 * y2),
                _ln((mx2 - mx1) / x2) * 5.0,
                _ln((my2 - my1) / y2) * 5.0,
            )
            if tail:
                crow = jnp.minimum(lrows, _TAIL - 1)
            else:
                crow = lrows
            s = _splat_f(0.0)
            lbase = crow * 4
            for a in range(4):
                ld = plsc.load_gather(lc_ref, [lbase + a])
                d = ld - enc[a]
                ad = jnp.abs(d)
                s = s + jnp.where(ad < 1.0, 0.5 * d * d, ad - 0.5)
            ll = ll + jnp.where(pos, s, 0.0)
            mx = _splat_f(-3.4e38)
            for j in range(_NCLS):
                c = plsc.load_gather(cf_ref, [crow, _splat_i(j)])
                mx = jnp.maximum(mx, c)
            ssum = _splat_f(0.0)
            gat = _splat_f(0.0)
            for j in range(_NCLS):
                c = plsc.load_gather(cf_ref, [crow, _splat_i(j)])
                ssum = ssum + jnp.exp(c - mx)
                gat = jnp.where(conf_t == j, c, gat)
            ce = _ln(ssum) + mx - gat
            sp = sp + jnp.where(pos, ce, 0.0)
            r_ref[sl] = jnp.where(pos | ~valid, 0.0, ce)
            return ll, sp, npn

        return b_body

    def chunk_body(ci, carry):
        cb = pl.multiple_of(ci * _CHUNK, _CHUNK)
        pltpu.sync_copy(conf_hbm.at[b, pl.ds(cb, _CHUNK)], conf_buf)
        pltpu.sync_copy(loc_hbm.at[pl.ds(b * (_P * 4) + cb * 4, _CHUNK * 4)],
                        loc_buf)
        return lax.fori_loop(0, _CHUNK // _L,
                             make_chunk(cb, conf_buf, loc_buf, _CHUNK // _L,
                                        False),
                             carry)

    ll_acc, spce, npos = lax.fori_loop(0, _NCHUNK, chunk_body,
                                       (ll_acc, spce, npos))

    tail_fn = make_chunk(_MAIN, ct_buf, lt_buf, 2, True)
    ll_acc, spce, npos = tail_fn(0, (ll_acc, spce, npos))
    ll_acc, spce, npos = tail_fn(1, (ll_acc, spce, npos))

    num_pos = jnp.sum(npos)
    k = jnp.minimum(3 * num_pos, _P - 1)

    def bs_body(i, lohi):
        lo, hi = lohi
        mid = lo + lax.shift_right_arithmetic(hi - lo + 1, 1)

        def cnt_body(v, c, mid=mid):
            bits = plsc.bitcast(r_ref[pl.ds(v * _L, _L)], jnp.int32)
            return c + plsc.all_reduce_population_count(bits >= mid)

        cnt = lax.fori_loop(0, _PV, cnt_body, jnp.zeros((_L,), jnp.int32))
        take = cnt >= k
        return jnp.where(take, mid, lo), jnp.where(take, hi, mid - 1)

    lo, hi = lax.fori_loop(0, 31, bs_body,
                           (jnp.zeros((_L,), jnp.int32),
                            _splat_i(_F32_MAX_BITS)))
    thr = plsc.bitcast(lo, jnp.float32)

    def tally_body(v, carry):
        cg, sg = carry
        rv = r_ref[pl.ds(v * _L, _L)]
        m = rv > thr
        return (cg + plsc.all_reduce_population_count(m),
                sg + jnp.where(m, rv, 0.0))

    cnt_gt, sum_gt = lax.fori_loop(0, _PV, tally_body,
                                   (jnp.zeros((_L,), jnp.int32),
                                    _splat_f(0.0)))
    extra = (k - cnt_gt).astype(jnp.float32)
    lc_vec = spce + sum_gt + jnp.where(iota == 0, extra * thr, 0.0)

    res_buf[0, pl.ds(0, _L)] = ll_acc
    res_buf[0, pl.ds(_L, _L)] = lc_vec
    res_buf[0, pl.ds(2 * _L, _L)] = npos.astype(jnp.float32)
    pltpu.sync_copy(res_buf, out_hbm.at[b])


_mbl = functools.partial(
    pl.kernel,
    out_type=jax.ShapeDtypeStruct((_B, 1, 3 * _L), jnp.float32),
    mesh=plsc.VectorSubcoreMesh(core_axis_name="c", subcore_axis_name="s"),
    compiler_params=pltpu.CompilerParams(use_tc_tiling_on_sc=True,
                                         needs_layout_passes=False),
    scratch_types=[
        pltpu.VMEM((_P * 4,), jnp.float32),
        pltpu.VMEM((_PPAD,), jnp.float32),
        pltpu.VMEM((_PPAD,), jnp.int32),
        pltpu.VMEM((_PPAD,), jnp.float32),
        pltpu.VMEM((_CHUNK, _NCLS), jnp.float32),
        pltpu.VMEM((_CHUNK * 4,), jnp.float32),
        pltpu.VMEM((_TAIL, _NCLS), jnp.float32),
        pltpu.VMEM((_TAIL * 4,), jnp.float32),
        pltpu.VMEM((8 + _NTRUTH, 5), jnp.float32),
        pltpu.VMEM((1, 3 * _L), jnp.float32),
    ],
)(_mbl_body)


def kernel(loc_data, conf_data, priors, targets):
    out = _mbl(loc_data.reshape(_B * _P * 4),
               conf_data,
               priors.reshape(_P * 4),
               targets)
    sums = jnp.sum(out.reshape(_B, 3, _L), axis=(0, 2))
    n = sums[2]
    return sums[0] / n, sums[1] / n
